# baseline (speedup 1.0000x reference)
import functools

import numpy as np
import jax
import jax.numpy as jnp
from jax import lax
from jax.experimental import pallas as pl
from jax.experimental.pallas import tpu as pltpu
from jax.experimental.pallas import tpu_sc as plsc

N_NODES = 10000
N_EDGES = 320000
D_IN = 128
H1 = 16
H2 = 256
ACT = 32
LOG_STD_MIN, LOG_STD_MAX = -20.0, 2.0

NC = 2
NS = 16
NW = NC * NS
CHUNK = 128
CH = 80
E_PAD = NW * CH * CHUNK
NBUF = 4
NOUT = CH // NBUF
N_PAD = 10240
RPT = N_PAD // NS

_LOG2 = float(np.log(2.0))
_HALF_LOG_2PI = float(0.5 * np.log(2.0 * np.pi))

_sc_mesh = plsc.VectorSubcoreMesh(core_axis_name="c", subcore_axis_name="s")
_sc_params = pltpu.CompilerParams(use_tc_tiling_on_sc=False)


ZB = 64


@functools.partial(
    pl.kernel,
    out_type=jax.ShapeDtypeStruct((NC, N_PAD, H1), jnp.float32),
    mesh=_sc_mesh,
    compiler_params=_sc_params,
    scratch_types=[
        pltpu.VMEM((CH, CHUNK), jnp.int32),
        pltpu.VMEM((CHUNK, H1), jnp.float32),
        pltpu.VMEM((ZB, H1), jnp.float32),
        pltpu.VMEM_SHARED((N_PAD, H1), jnp.float32),
        pltpu.SemaphoreType.DMA,
    ],
)
def _sc_count(col_hbm, out_hbm, colv, onesv, zbuf, acc, sem):
    cid = lax.axis_index("c")
    sid = lax.axis_index("s")
    wid = cid * NS + sid

    def fill_body(i, carry):
        zbuf[i] = jnp.zeros((H1,), jnp.float32)
        return carry

    lax.fori_loop(0, ZB, fill_body, 0)

    def ones_body(i, carry):
        onesv[i] = jnp.full((H1,), 1.0, jnp.float32)
        return carry

    lax.fori_loop(0, CHUNK, ones_body, 0)

    def zero_body(z, carry):
        pltpu.sync_copy(zbuf, acc.at[pl.ds(sid * RPT + z * ZB, ZB)])
        return carry

    lax.fori_loop(0, RPT // ZB, zero_body, 0)
    pltpu.sync_copy(col_hbm.at[wid], colv)
    plsc.subcore_barrier()

    KF = 8

    def body(jo, carry):
        for k in range(KF):
            pltpu.async_copy(onesv, acc.at[colv.at[jo * KF + k]], sem,
                             add=True)
        for k in range(KF):
            pltpu.make_async_copy(onesv, acc.at[colv.at[jo * KF + k]],
                                  sem).wait()
        return carry

    lax.fori_loop(0, CH // KF, body, 0)
    plsc.subcore_barrier()

    def rb_body(z, carry):
        pltpu.sync_copy(acc.at[pl.ds(sid * RPT + z * ZB, ZB)], zbuf)
        pltpu.sync_copy(zbuf, out_hbm.at[cid, pl.ds(sid * RPT + z * ZB, ZB)])
        return carry

    lax.fori_loop(0, RPT // ZB, rb_body, 0)


@functools.partial(
    pl.kernel,
    out_type=jax.ShapeDtypeStruct((NC, N_PAD, H1), jnp.float32),
    mesh=_sc_mesh,
    compiler_params=_sc_params,
    scratch_types=[
        pltpu.VMEM((CH, CHUNK), jnp.int32),
        pltpu.VMEM((CH, CHUNK), jnp.int32),
        pltpu.VMEM((NBUF, CHUNK, H1), jnp.float32),
        pltpu.VMEM((ZB, H1), jnp.float32),
        pltpu.VMEM_SHARED((N_PAD, H1), jnp.float32),
        pltpu.SemaphoreType.DMA,
        pltpu.SemaphoreType.DMA,
        pltpu.SemaphoreType.DMA,
        pltpu.SemaphoreType.DMA,
        pltpu.SemaphoreType.DMA,
        pltpu.SemaphoreType.DMA,
        pltpu.SemaphoreType.DMA,
        pltpu.SemaphoreType.DMA,
    ],
)
def _sc_scatter(vals_hbm, row_hbm, col_hbm, out_hbm,
                rowv, colv, gbuf, zbuf, acc,
                gsem0, gsem1, gsem2, gsem3, ssem0, ssem1, ssem2, ssem3):
    cid = lax.axis_index("c")
    sid = lax.axis_index("s")
    wid = cid * NS + sid
    gsems = (gsem0, gsem1, gsem2, gsem3)
    ssems = (ssem0, ssem1, ssem2, ssem3)

    def _gather_start(g, b):
        pltpu.async_copy(vals_hbm.at[rowv.at[g]], gbuf.at[b], gsems[b])

    def _gather_wait(g, b):
        pltpu.make_async_copy(
            vals_hbm.at[rowv.at[g]], gbuf.at[b], gsems[b]).wait()

    def _scatter_start(g, b):
        pltpu.async_copy(gbuf.at[b], acc.at[colv.at[g]], ssems[b], add=True)

    def _scatter_wait(g, b):
        pltpu.make_async_copy(
            gbuf.at[b], acc.at[colv.at[g]], ssems[b]).wait()

    def fill_body(i, carry):
        zbuf[i] = jnp.zeros((H1,), jnp.float32)
        return carry

    lax.fori_loop(0, ZB, fill_body, 0)

    def zero_body(z, carry):
        pltpu.sync_copy(zbuf, acc.at[pl.ds(sid * RPT + z * ZB, ZB)])
        return carry

    lax.fori_loop(0, RPT // ZB, zero_body, 0)
    pltpu.sync_copy(row_hbm.at[wid], rowv)
    pltpu.sync_copy(col_hbm.at[wid], colv)
    plsc.subcore_barrier()

    for b in range(NBUF):
        _gather_start(b, b)

    def body(go, carry):
        for b in range(NBUF):
            g = go * NBUF + b
            _gather_wait(g, b)
            _scatter_start(g, b)

            @pl.when(go < NOUT - 1)
            def _refill():
                _scatter_wait(g, b)
                _gather_start(g + NBUF, b)

        return carry

    lax.fori_loop(0, NOUT, body, 0)
    for b in range(NBUF):
        _scatter_wait((NOUT - 1) * NBUF + b, b)
    plsc.subcore_barrier()

    def rb_body(z, carry):
        pltpu.sync_copy(acc.at[pl.ds(sid * RPT + z * ZB, ZB)], zbuf)
        pltpu.sync_copy(zbuf, out_hbm.at[cid, pl.ds(sid * RPT + z * ZB, ZB)])
        return carry

    lax.fori_loop(0, RPT // ZB, rb_body, 0)


TB = 1000


def _tc_mm_body(obs_ref, w1_ref, p1_ref):
    p1_ref[...] = jnp.dot(obs_ref[...], w1_ref[...],
                          preferred_element_type=jnp.float32)


def _tc_mm(obs, W1):
    blk = lambda i: (i, 0)
    full = lambda i: (0, 0)
    return pl.pallas_call(
        _tc_mm_body,
        grid=(N_NODES // TB,),
        in_specs=[
            pl.BlockSpec((TB, D_IN), blk),
            pl.BlockSpec((D_IN, H1), full),
        ],
        out_specs=pl.BlockSpec((TB, H1), blk),
        out_shape=jax.ShapeDtypeStruct((N_PAD, H1), jnp.float32),
    )(obs, W1)


def _tc_prep_body(p1_ref, cnt_ref, p1p_ref, dinv_ref):
    dinv = lax.rsqrt(1.0 + cnt_ref[0] + cnt_ref[1])
    p1p_ref[...] = p1_ref[...] * dinv
    dinv_ref[...] = dinv


def _tc_prep(p1, cnt):
    blk = lambda i: (i, 0)
    stk = lambda i: (0, i, 0)
    return pl.pallas_call(
        _tc_prep_body,
        grid=(N_NODES // TB,),
        in_specs=[
            pl.BlockSpec((TB, H1), blk),
            pl.BlockSpec((NC, TB, H1), stk),
        ],
        out_specs=[
            pl.BlockSpec((TB, H1), blk),
            pl.BlockSpec((TB, H1), blk),
        ],
        out_shape=[
            jax.ShapeDtypeStruct((N_PAD, H1), jnp.float32),
            jax.ShapeDtypeStruct((N_PAD, H1), jnp.float32),
        ],
    )(p1, cnt)


def _tc_mid_body(t1_ref, p1p_ref, dinv_ref, b1_ref, p2p_ref):
    dinv = dinv_ref[...]
    h1 = jax.nn.relu(dinv * (t1_ref[0] + t1_ref[1] + p1p_ref[...])
                     + b1_ref[...])
    p2p_ref[...] = dinv * h1


def _tc_mid(t1, p1p, dinv16, b1r):
    blk = lambda i: (i, 0)
    full = lambda i: (0, 0)
    stk = lambda i: (0, i, 0)
    return pl.pallas_call(
        _tc_mid_body,
        grid=(N_NODES // TB,),
        in_specs=[
            pl.BlockSpec((NC, TB, H1), stk),
            pl.BlockSpec((TB, H1), blk),
            pl.BlockSpec((TB, H1), blk),
            pl.BlockSpec((1, H1), full),
        ],
        out_specs=pl.BlockSpec((TB, H1), blk),
        out_shape=jax.ShapeDtypeStruct((N_PAD, H1), jnp.float32),
    )(t1, p1p, dinv16, b1r)


def _tc_head_body(t2_ref, p2p_ref, dinv_ref, eps_ref,
                  w2_ref, b2_ref, m1w_ref, m1b_ref, m2w_ref, m2b_ref,
                  m3w_ref, m3b_ref, act_ref, lp_ref):
    x = dinv_ref[...] * (t2_ref[0] + t2_ref[1] + p2p_ref[...])
    bf = jnp.bfloat16
    h2 = jnp.dot(x.astype(bf), w2_ref[...].astype(bf),
                 preferred_element_type=jnp.float32) + b2_ref[...]
    z = jax.nn.relu(
        jnp.dot(h2.astype(bf), m1w_ref[...].astype(bf),
                preferred_element_type=jnp.float32) + m1b_ref[...])
    z = jax.nn.relu(
        jnp.dot(z.astype(bf), m2w_ref[...].astype(bf),
                preferred_element_type=jnp.float32) + m2b_ref[...])
    o = jnp.dot(z.astype(bf), m3w_ref[...].astype(bf),
                preferred_element_type=jnp.float32) + m3b_ref[...]
    mean = o[:, :ACT]
    log_std = jnp.clip(o[:, ACT:], LOG_STD_MIN, LOG_STD_MAX)
    std = jnp.exp(log_std)
    eps = eps_ref[...]
    pre = mean + std * eps
    act_ref[...] = jnp.tanh(pre)
    a2 = jnp.abs(2.0 * pre)
    lp = (-0.5 * eps * eps - log_std - _HALF_LOG_2PI - 2.0 * _LOG2
          + a2 + 2.0 * jnp.log1p(jnp.exp(-a2)))
    lp_ref[...] = jnp.sum(lp, axis=1, keepdims=True)


def _tc_head(t2, p2p, dinv16, eps, W2, b2r, M1w, M1br, M2w, M2br,
             M3w, M3br):
    blk = lambda i: (i, 0)
    full = lambda i: (0, 0)
    stk = lambda i: (0, i, 0)
    return pl.pallas_call(
        _tc_head_body,
        grid=(N_NODES // TB,),
        in_specs=[
            pl.BlockSpec((NC, TB, H1), stk),
            pl.BlockSpec((TB, H1), blk),
            pl.BlockSpec((TB, H1), blk),
            pl.BlockSpec((TB, ACT), blk),
            pl.BlockSpec((H1, H2), full),
            pl.BlockSpec((1, H2), full),
            pl.BlockSpec((H2, 256), full),
            pl.BlockSpec((1, 256), full),
            pl.BlockSpec((256, 256), full),
            pl.BlockSpec((1, 256), full),
            pl.BlockSpec((256, 2 * ACT), full),
            pl.BlockSpec((1, 2 * ACT), full),
        ],
        out_specs=[
            pl.BlockSpec((TB, ACT), blk),
            pl.BlockSpec((TB, 1), blk),
        ],
        out_shape=[
            jax.ShapeDtypeStruct((N_NODES, ACT), jnp.float32),
            jax.ShapeDtypeStruct((N_NODES, 1), jnp.float32),
        ],
    )(t2, p2p, dinv16, eps, W2, b2r, M1w, M1br, M2w, M2br, M3w, M3br)


def kernel(obs, edge_index, eps, W1, b1, W2, b2,
           M1w, M1b, M2w, M2b, M3w, M3b):
    ei = edge_index.astype(jnp.int32)
    pad = jnp.full((E_PAD - N_EDGES,), N_NODES, jnp.int32)
    row = jnp.concatenate([ei[0], pad]).reshape(NW, CH, CHUNK)
    col = jnp.concatenate([ei[1], pad]).reshape(NW, CH, CHUNK)

    p1 = _tc_mm(obs, W1)
    cnt = _sc_count(col)
    p1p, dinv16 = _tc_prep(p1, cnt)
    t1 = _sc_scatter(p1p, row, col)
    p2p = _tc_mid(t1, p1p, dinv16, b1.reshape(1, H1))
    t2 = _sc_scatter(p2p, row, col)
    action, log_prob = _tc_head(
        t2, p2p, dinv16, eps, W2, b2.reshape(1, H2), M1w, M1b.reshape(1, 256),
        M2w, M2b.reshape(1, 256), M3w, M3b.reshape(1, 2 * ACT))
    return (action, log_prob)


# speedup vs baseline: 1.0297x; 1.0297x over previous
import functools

import numpy as np
import jax
import jax.numpy as jnp
from jax import lax
from jax.experimental import pallas as pl
from jax.experimental.pallas import tpu as pltpu
from jax.experimental.pallas import tpu_sc as plsc

N_NODES = 10000
N_EDGES = 320000
D_IN = 128
H1 = 16
H2 = 256
ACT = 32
LOG_STD_MIN, LOG_STD_MAX = -20.0, 2.0

NC = 2
NS = 16
NW = NC * NS
CHUNK = 128
CH = 80
E_PAD = NW * CH * CHUNK
NBUF = 4
NOUT = CH // NBUF
N_PAD = 10240
RPT = N_PAD // NS

_LOG2 = float(np.log(2.0))
_HALF_LOG_2PI = float(0.5 * np.log(2.0 * np.pi))

_sc_mesh = plsc.VectorSubcoreMesh(core_axis_name="c", subcore_axis_name="s")
_sc_params = pltpu.CompilerParams(use_tc_tiling_on_sc=False)


ZB = 64


@functools.partial(
    pl.kernel,
    out_type=jax.ShapeDtypeStruct((NC, N_PAD, H1), jnp.float32),
    mesh=_sc_mesh,
    compiler_params=_sc_params,
    scratch_types=[
        pltpu.VMEM((CH, CHUNK), jnp.int32),
        pltpu.VMEM((CHUNK, H1), jnp.float32),
        pltpu.VMEM((ZB, H1), jnp.float32),
        pltpu.VMEM_SHARED((N_PAD, H1), jnp.float32),
        pltpu.SemaphoreType.DMA,
    ],
)
def _sc_count(col_hbm, out_hbm, colv, onesv, zbuf, acc, sem):
    cid = lax.axis_index("c")
    sid = lax.axis_index("s")
    wid = cid * NS + sid

    def fill_body(i, carry):
        zbuf[i] = jnp.zeros((H1,), jnp.float32)
        return carry

    lax.fori_loop(0, ZB, fill_body, 0)

    def ones_body(i, carry):
        onesv[i] = jnp.full((H1,), 1.0, jnp.float32)
        return carry

    lax.fori_loop(0, CHUNK, ones_body, 0)

    def zero_body(z, carry):
        pltpu.sync_copy(zbuf, acc.at[pl.ds(sid * RPT + z * ZB, ZB)])
        return carry

    lax.fori_loop(0, RPT // ZB, zero_body, 0)
    pltpu.sync_copy(col_hbm.at[wid], colv)
    plsc.subcore_barrier()

    KF = 8

    def body(jo, carry):
        for k in range(KF):
            pltpu.async_copy(onesv, acc.at[colv.at[jo * KF + k]], sem,
                             add=True)
        for k in range(KF):
            pltpu.make_async_copy(onesv, acc.at[colv.at[jo * KF + k]],
                                  sem).wait()
        return carry

    lax.fori_loop(0, CH // KF, body, 0)
    plsc.subcore_barrier()

    def rb_body(z, carry):
        pltpu.sync_copy(acc.at[pl.ds(sid * RPT + z * ZB, ZB)], zbuf)
        pltpu.sync_copy(zbuf, out_hbm.at[cid, pl.ds(sid * RPT + z * ZB, ZB)])
        return carry

    lax.fori_loop(0, RPT // ZB, rb_body, 0)


@functools.partial(
    pl.kernel,
    out_type=jax.ShapeDtypeStruct((NC, N_PAD, H1), jnp.float32),
    mesh=_sc_mesh,
    compiler_params=_sc_params,
    scratch_types=[
        pltpu.VMEM((CH, CHUNK), jnp.int32),
        pltpu.VMEM((CH, CHUNK), jnp.int32),
        pltpu.VMEM((NBUF, CHUNK, H1), jnp.float32),
        pltpu.VMEM((ZB, H1), jnp.float32),
        pltpu.VMEM_SHARED((N_PAD, H1), jnp.float32),
        pltpu.SemaphoreType.DMA,
        pltpu.SemaphoreType.DMA,
        pltpu.SemaphoreType.DMA,
        pltpu.SemaphoreType.DMA,
        pltpu.SemaphoreType.DMA,
        pltpu.SemaphoreType.DMA,
        pltpu.SemaphoreType.DMA,
        pltpu.SemaphoreType.DMA,
    ],
)
def _sc_scatter(vals_hbm, row_hbm, col_hbm, out_hbm,
                rowv, colv, gbuf, zbuf, acc,
                gsem0, gsem1, gsem2, gsem3, ssem0, ssem1, ssem2, ssem3):
    cid = lax.axis_index("c")
    sid = lax.axis_index("s")
    wid = cid * NS + sid
    gsems = (gsem0, gsem1, gsem2, gsem3)
    ssems = (ssem0, ssem1, ssem2, ssem3)

    def _gather_start(g, b):
        pltpu.async_copy(vals_hbm.at[rowv.at[g]], gbuf.at[b], gsems[b])

    def _gather_wait(g, b):
        pltpu.make_async_copy(
            vals_hbm.at[rowv.at[g]], gbuf.at[b], gsems[b]).wait()

    def _scatter_start(g, b):
        pltpu.async_copy(gbuf.at[b], acc.at[colv.at[g]], ssems[b], add=True)

    def _scatter_wait(g, b):
        pltpu.make_async_copy(
            gbuf.at[b], acc.at[colv.at[g]], ssems[b]).wait()

    def fill_body(i, carry):
        zbuf[i] = jnp.zeros((H1,), jnp.float32)
        return carry

    lax.fori_loop(0, ZB, fill_body, 0)

    def zero_body(z, carry):
        pltpu.sync_copy(zbuf, acc.at[pl.ds(sid * RPT + z * ZB, ZB)])
        return carry

    lax.fori_loop(0, RPT // ZB, zero_body, 0)
    pltpu.sync_copy(row_hbm.at[wid], rowv)
    pltpu.sync_copy(col_hbm.at[wid], colv)
    plsc.subcore_barrier()

    for b in range(NBUF):
        _gather_start(b, b)

    def body(go, carry):
        for b in range(NBUF):
            g = go * NBUF + b
            _gather_wait(g, b)
            _scatter_start(g, b)

            @pl.when(go < NOUT - 1)
            def _refill():
                _scatter_wait(g, b)
                _gather_start(g + NBUF, b)

        return carry

    lax.fori_loop(0, NOUT, body, 0)
    for b in range(NBUF):
        _scatter_wait((NOUT - 1) * NBUF + b, b)
    plsc.subcore_barrier()

    def rb_body(z, carry):
        pltpu.sync_copy(acc.at[pl.ds(sid * RPT + z * ZB, ZB)], zbuf)
        pltpu.sync_copy(zbuf, out_hbm.at[cid, pl.ds(sid * RPT + z * ZB, ZB)])
        return carry

    lax.fori_loop(0, RPT // ZB, rb_body, 0)


TB = 1000


def _tc_prep_body(obs_ref, w1_ref, cnt_ref, p1p_ref, dinv_ref):
    dinv = lax.rsqrt(1.0 + cnt_ref[0] + cnt_ref[1])
    p1 = jnp.dot(obs_ref[...], w1_ref[...], preferred_element_type=jnp.float32)
    p1p_ref[...] = p1 * dinv
    dinv_ref[...] = dinv


def _tc_prep(obs, W1, cnt):
    blk = lambda i: (i, 0)
    full = lambda i: (0, 0)
    stk = lambda i: (0, i, 0)
    return pl.pallas_call(
        _tc_prep_body,
        grid=(N_NODES // TB,),
        in_specs=[
            pl.BlockSpec((TB, D_IN), blk),
            pl.BlockSpec((D_IN, H1), full),
            pl.BlockSpec((NC, TB, H1), stk),
        ],
        out_specs=[
            pl.BlockSpec((TB, H1), blk),
            pl.BlockSpec((TB, H1), blk),
        ],
        out_shape=[
            jax.ShapeDtypeStruct((N_PAD, H1), jnp.float32),
            jax.ShapeDtypeStruct((N_PAD, H1), jnp.float32),
        ],
    )(obs, W1, cnt)


def _tc_mid_body(t1_ref, p1p_ref, dinv_ref, b1_ref, p2p_ref):
    dinv = dinv_ref[...]
    h1 = jax.nn.relu(dinv * (t1_ref[0] + t1_ref[1] + p1p_ref[...])
                     + b1_ref[...])
    p2p_ref[...] = dinv * h1


def _tc_mid(t1, p1p, dinv16, b1r):
    blk = lambda i: (i, 0)
    full = lambda i: (0, 0)
    stk = lambda i: (0, i, 0)
    return pl.pallas_call(
        _tc_mid_body,
        grid=(N_NODES // TB,),
        in_specs=[
            pl.BlockSpec((NC, TB, H1), stk),
            pl.BlockSpec((TB, H1), blk),
            pl.BlockSpec((TB, H1), blk),
            pl.BlockSpec((1, H1), full),
        ],
        out_specs=pl.BlockSpec((TB, H1), blk),
        out_shape=jax.ShapeDtypeStruct((N_PAD, H1), jnp.float32),
    )(t1, p1p, dinv16, b1r)


def _tc_head_body(t2_ref, p2p_ref, dinv_ref, eps_ref,
                  w2_ref, b2_ref, m1w_ref, m1b_ref, m2w_ref, m2b_ref,
                  m3w_ref, m3b_ref, act_ref, lp_ref):
    x = dinv_ref[...] * (t2_ref[0] + t2_ref[1] + p2p_ref[...])
    bf = jnp.bfloat16
    h2 = jnp.dot(x.astype(bf), w2_ref[...].astype(bf),
                 preferred_element_type=jnp.float32) + b2_ref[...]
    z = jax.nn.relu(
        jnp.dot(h2.astype(bf), m1w_ref[...].astype(bf),
                preferred_element_type=jnp.float32) + m1b_ref[...])
    z = jax.nn.relu(
        jnp.dot(z.astype(bf), m2w_ref[...].astype(bf),
                preferred_element_type=jnp.float32) + m2b_ref[...])
    o = jnp.dot(z.astype(bf), m3w_ref[...].astype(bf),
                preferred_element_type=jnp.float32) + m3b_ref[...]
    mean = o[:, :ACT]
    log_std = jnp.clip(o[:, ACT:], LOG_STD_MIN, LOG_STD_MAX)
    std = jnp.exp(log_std)
    eps = eps_ref[...]
    pre = mean + std * eps
    act_ref[...] = jnp.tanh(pre)
    a2 = jnp.abs(2.0 * pre)
    lp = (-0.5 * eps * eps - log_std - _HALF_LOG_2PI - 2.0 * _LOG2
          + a2 + 2.0 * jnp.log1p(jnp.exp(-a2)))
    lp_ref[...] = jnp.sum(lp, axis=1, keepdims=True)


def _tc_head(t2, p2p, dinv16, eps, W2, b2r, M1w, M1br, M2w, M2br,
             M3w, M3br):
    blk = lambda i: (i, 0)
    full = lambda i: (0, 0)
    stk = lambda i: (0, i, 0)
    return pl.pallas_call(
        _tc_head_body,
        grid=(N_NODES // TB,),
        in_specs=[
            pl.BlockSpec((NC, TB, H1), stk),
            pl.BlockSpec((TB, H1), blk),
            pl.BlockSpec((TB, H1), blk),
            pl.BlockSpec((TB, ACT), blk),
            pl.BlockSpec((H1, H2), full),
            pl.BlockSpec((1, H2), full),
            pl.BlockSpec((H2, 256), full),
            pl.BlockSpec((1, 256), full),
            pl.BlockSpec((256, 256), full),
            pl.BlockSpec((1, 256), full),
            pl.BlockSpec((256, 2 * ACT), full),
            pl.BlockSpec((1, 2 * ACT), full),
        ],
        out_specs=[
            pl.BlockSpec((TB, ACT), blk),
            pl.BlockSpec((TB, 1), blk),
        ],
        out_shape=[
            jax.ShapeDtypeStruct((N_NODES, ACT), jnp.float32),
            jax.ShapeDtypeStruct((N_NODES, 1), jnp.float32),
        ],
    )(t2, p2p, dinv16, eps, W2, b2r, M1w, M1br, M2w, M2br, M3w, M3br)


def kernel(obs, edge_index, eps, W1, b1, W2, b2,
           M1w, M1b, M2w, M2b, M3w, M3b):
    ei = edge_index.astype(jnp.int32)
    pad = jnp.full((E_PAD - N_EDGES,), N_NODES, jnp.int32)
    row = jnp.concatenate([ei[0], pad]).reshape(NW, CH, CHUNK)
    col = jnp.concatenate([ei[1], pad]).reshape(NW, CH, CHUNK)

    cnt = _sc_count(col)
    p1p, dinv16 = _tc_prep(obs, W1, cnt)
    t1 = _sc_scatter(p1p, row, col)
    p2p = _tc_mid(t1, p1p, dinv16, b1.reshape(1, H1))
    t2 = _sc_scatter(p2p, row, col)
    action, log_prob = _tc_head(
        t2, p2p, dinv16, eps, W2, b2.reshape(1, H2), M1w, M1b.reshape(1, 256),
        M2w, M2b.reshape(1, 256), M3w, M3b.reshape(1, 2 * ACT))
    return (action, log_prob)
